# baseline (speedup 1.0000x reference)
import functools

import jax
import jax.numpy as jnp
from jax import lax
from jax.experimental import pallas as pl
from jax.experimental.pallas import tpu as pltpu
from jax.experimental.pallas import tpu_sc as plsc

N_NODES = 10000
N_EDGES = 320000
F = 128

NC = 2
NS = 16
EDGES_PER_CORE = N_EDGES // NC
EDGES_PER_TILE = EDGES_PER_CORE // NS
EB = 80
N_BATCH = EDGES_PER_TILE // EB
ACC_ROWS = 10240
ROWS_PER_TILE = ACC_ROWS // NS
ZR = 32


def _sc_scatter(x, ei):
    mesh = plsc.VectorSubcoreMesh(
        core_axis_name="c", subcore_axis_name="s",
        num_cores=NC, num_subcores=NS)

    @functools.partial(
        pl.kernel,
        out_type=jax.ShapeDtypeStruct((NC, ACC_ROWS, F), jnp.float32),
        mesh=mesh,
        scratch_types=[
            pltpu.VMEM((N_BATCH * EB,), jnp.int32),
            pltpu.VMEM((N_BATCH * EB,), jnp.int32),
            pltpu.VMEM((EB, F), jnp.float32),
            pltpu.VMEM((EB, F), jnp.float32),
            pltpu.VMEM((ZR, F), jnp.float32),
            pltpu.VMEM_SHARED((ACC_ROWS, F), jnp.float32),
            pltpu.SemaphoreType.DMA,
            pltpu.SemaphoreType.DMA,
            pltpu.SemaphoreType.DMA,
        ],
        compiler_params=pltpu.CompilerParams(use_tc_tiling_on_sc=False),
    )
    def k(x_hbm, ei_hbm, out_hbm, cidx_v, ridx_v, buf_a, buf_b,
          zbuf_v, acc_sh, sem_a, sem_b, sem_z):
        c = lax.axis_index("c")
        s = lax.axis_index("s")

        tid = c * NS + s
        idx_cp_c = pltpu.async_copy(
            ei_hbm.at[1, pl.ds(tid * EDGES_PER_TILE, EDGES_PER_TILE)],
            cidx_v, sem_a)
        idx_cp_r = pltpu.async_copy(
            ei_hbm.at[0, pl.ds(tid * EDGES_PER_TILE, EDGES_PER_TILE)],
            ridx_v, sem_b)

        zero16 = jnp.zeros((16,), jnp.float32)
        for i in range(ZR):
            for j in range(F // 16):
                zbuf_v[i, pl.ds(j * 16, 16)] = zero16

        base_row = s * ROWS_PER_TILE

        def zfire(i, carry):
            pltpu.async_copy(zbuf_v, acc_sh.at[pl.ds(base_row + i * ZR, ZR)],
                             sem_z)
            return carry

        def zdrain(i, carry):
            pltpu.make_async_copy(
                zbuf_v, acc_sh.at[pl.ds(base_row + i * ZR, ZR)],
                sem_z).wait()
            return carry

        def gather(i, buf, sem):
            pltpu.async_copy(x_hbm.at[cidx_v.at[pl.ds(i * EB, EB)]], buf, sem)

        def gwait(i, buf, sem):
            pltpu.make_async_copy(
                x_hbm.at[cidx_v.at[pl.ds(i * EB, EB)]], buf, sem).wait()

        def scat(i, buf):
            pltpu.sync_copy(buf, acc_sh.at[ridx_v.at[pl.ds(i * EB, EB)]],
                            add=True)

        lax.fori_loop(0, ROWS_PER_TILE // ZR, zfire, 0)
        idx_cp_c.wait()
        idx_cp_r.wait()
        gather(0, buf_a, sem_a)
        gather(1, buf_b, sem_b)
        lax.fori_loop(0, ROWS_PER_TILE // ZR, zdrain, 0)
        plsc.subcore_barrier()

        def eloop(j, carry):
            i = 2 * j
            gwait(i, buf_a, sem_a)
            scat(i, buf_a)
            gather(i + 2, buf_a, sem_a)
            gwait(i + 1, buf_b, sem_b)
            scat(i + 1, buf_b)
            gather(i + 3, buf_b, sem_b)
            return carry

        n_even = (N_BATCH // 2) * 2
        lax.fori_loop(0, n_even // 2 - 1, eloop, 0)
        gwait(n_even - 2, buf_a, sem_a)
        scat(n_even - 2, buf_a)
        gwait(n_even - 1, buf_b, sem_b)
        scat(n_even - 1, buf_b)
        for i in range(n_even, N_BATCH):
            gather(i, buf_a, sem_a)
            gwait(i, buf_a, sem_a)
            scat(i, buf_a)
        plsc.subcore_barrier()

        pltpu.sync_copy(acc_sh.at[pl.ds(base_row, ROWS_PER_TILE)],
                        out_hbm.at[c].at[pl.ds(base_row, ROWS_PER_TILE)])

    return k(x, ei)


def _tc_combine(partials, W, b2d):
    BR = 1280

    def body(p_ref, w_ref, b_ref, o_ref):
        agg = p_ref[0] + p_ref[1]
        o_ref[...] = jnp.dot(agg, w_ref[...],
                             preferred_element_type=jnp.float32) + b_ref[...]

    return pl.pallas_call(
        body,
        grid=(ACC_ROWS // BR,),
        in_specs=[
            pl.BlockSpec((NC, BR, F), lambda i: (0, i, 0)),
            pl.BlockSpec((F, F), lambda i: (0, 0)),
            pl.BlockSpec((1, F), lambda i: (0, 0)),
        ],
        out_specs=pl.BlockSpec((BR, F), lambda i: (i, 0)),
        out_shape=jax.ShapeDtypeStruct((N_NODES, F), jnp.float32),
    )(partials, W, b2d)


def kernel(x, edge_index_or_adj, W, b):
    ei = edge_index_or_adj.astype(jnp.int32)
    partials = _sc_scatter(x, ei)
    return _tc_combine(partials, W, b.reshape(1, F))


# speedup vs baseline: 1.0143x; 1.0143x over previous
import functools

import jax
import jax.numpy as jnp
from jax import lax
from jax.experimental import pallas as pl
from jax.experimental.pallas import tpu as pltpu
from jax.experimental.pallas import tpu_sc as plsc

N_NODES = 10000
N_EDGES = 320000
F = 128

NC = 2
NS = 16
EDGES_PER_CORE = N_EDGES // NC
EDGES_PER_TILE = EDGES_PER_CORE // NS
EB = 80
N_BATCH = EDGES_PER_TILE // EB
ACC_ROWS = 10240
ROWS_PER_TILE = ACC_ROWS // NS
ZR = 8


def _sc_scatter(x, ei):
    mesh = plsc.VectorSubcoreMesh(
        core_axis_name="c", subcore_axis_name="s",
        num_cores=NC, num_subcores=NS)

    @functools.partial(
        pl.kernel,
        out_type=jax.ShapeDtypeStruct((NC, ACC_ROWS, F), jnp.float32),
        mesh=mesh,
        scratch_types=[
            pltpu.VMEM((N_BATCH * EB,), jnp.int32),
            pltpu.VMEM((N_BATCH * EB,), jnp.int32),
            pltpu.VMEM((EB, F), jnp.float32),
            pltpu.VMEM((EB, F), jnp.float32),
            pltpu.VMEM((ZR, F), jnp.float32),
            pltpu.VMEM_SHARED((ACC_ROWS, F), jnp.float32),
            pltpu.SemaphoreType.DMA,
            pltpu.SemaphoreType.DMA,
            pltpu.SemaphoreType.DMA,
        ],
        compiler_params=pltpu.CompilerParams(use_tc_tiling_on_sc=False),
    )
    def k(x_hbm, ei_hbm, out_hbm, cidx_v, ridx_v, buf_a, buf_b,
          zbuf_v, acc_sh, sem_a, sem_b, sem_z):
        c = lax.axis_index("c")
        s = lax.axis_index("s")

        tid = c * NS + s
        idx_cp_c = pltpu.async_copy(
            ei_hbm.at[1, pl.ds(tid * EDGES_PER_TILE, EDGES_PER_TILE)],
            cidx_v, sem_a)
        idx_cp_r = pltpu.async_copy(
            ei_hbm.at[0, pl.ds(tid * EDGES_PER_TILE, EDGES_PER_TILE)],
            ridx_v, sem_b)

        zero16 = jnp.zeros((16,), jnp.float32)
        for i in range(ZR):
            for j in range(F // 16):
                zbuf_v[i, pl.ds(j * 16, 16)] = zero16

        base_row = s * ROWS_PER_TILE

        def zfire(i, carry):
            pltpu.async_copy(zbuf_v, acc_sh.at[pl.ds(base_row + i * ZR, ZR)],
                             sem_z)
            return carry

        def zdrain(i, carry):
            pltpu.make_async_copy(
                zbuf_v, acc_sh.at[pl.ds(base_row + i * ZR, ZR)],
                sem_z).wait()
            return carry

        def gather(i, buf, sem):
            pltpu.async_copy(x_hbm.at[cidx_v.at[pl.ds(i * EB, EB)]], buf, sem)

        def gwait(i, buf, sem):
            pltpu.make_async_copy(
                x_hbm.at[cidx_v.at[pl.ds(i * EB, EB)]], buf, sem).wait()

        def scat(i, buf):
            pltpu.sync_copy(buf, acc_sh.at[ridx_v.at[pl.ds(i * EB, EB)]],
                            add=True)

        lax.fori_loop(0, ROWS_PER_TILE // ZR, zfire, 0)
        idx_cp_c.wait()
        idx_cp_r.wait()
        gather(0, buf_a, sem_a)
        gather(1, buf_b, sem_b)
        lax.fori_loop(0, ROWS_PER_TILE // ZR, zdrain, 0)
        plsc.subcore_barrier()

        def eloop(j, carry):
            i = 2 * j
            gwait(i, buf_a, sem_a)
            scat(i, buf_a)
            gather(i + 2, buf_a, sem_a)
            gwait(i + 1, buf_b, sem_b)
            scat(i + 1, buf_b)
            gather(i + 3, buf_b, sem_b)
            return carry

        n_even = (N_BATCH // 2) * 2
        lax.fori_loop(0, n_even // 2 - 1, eloop, 0)
        gwait(n_even - 2, buf_a, sem_a)
        scat(n_even - 2, buf_a)
        gwait(n_even - 1, buf_b, sem_b)
        scat(n_even - 1, buf_b)
        for i in range(n_even, N_BATCH):
            gather(i, buf_a, sem_a)
            gwait(i, buf_a, sem_a)
            scat(i, buf_a)
        plsc.subcore_barrier()

        pltpu.sync_copy(acc_sh.at[pl.ds(base_row, ROWS_PER_TILE)],
                        out_hbm.at[c].at[pl.ds(base_row, ROWS_PER_TILE)])

    return k(x, ei)


def _tc_combine(partials, W, b2d):
    BR = 2560

    def body(p_ref, w_ref, b_ref, o_ref):
        agg = p_ref[0] + p_ref[1]
        o_ref[...] = jnp.dot(agg, w_ref[...],
                             preferred_element_type=jnp.float32) + b_ref[...]

    return pl.pallas_call(
        body,
        grid=(ACC_ROWS // BR,),
        in_specs=[
            pl.BlockSpec((NC, BR, F), lambda i: (0, i, 0)),
            pl.BlockSpec((F, F), lambda i: (0, 0)),
            pl.BlockSpec((1, F), lambda i: (0, 0)),
        ],
        out_specs=pl.BlockSpec((BR, F), lambda i: (i, 0)),
        out_shape=jax.ShapeDtypeStruct((N_NODES, F), jnp.float32),
    )(partials, W, b2d)


def kernel(x, edge_index_or_adj, W, b):
    ei = edge_index_or_adj.astype(jnp.int32)
    partials = _sc_scatter(x, ei)
    return _tc_combine(partials, W, b.reshape(1, F))
